# baseline (speedup 1.0000x reference)
import jax
import jax.numpy as jnp
from jax.experimental import pallas as pl
from jax.experimental.pallas import tpu as pltpu

CHUNK = 512
NBUF = 4


def _pe_add_manual(x_hbm, pe_hbm, o_hbm, xbuf, obuf, pebuf, lsem, ssem, pesem):
    n, d = x_hbm.shape
    s = pe_hbm.shape[0]
    nchunks = n // CHUNK

    pltpu.make_async_copy(pe_hbm, pebuf, pesem).start()
    for b in range(min(NBUF, nchunks)):
        pltpu.make_async_copy(
            x_hbm.at[pl.ds(b * CHUNK, CHUNK), :], xbuf.at[b], lsem.at[b]
        ).start()
    pltpu.make_async_copy(pe_hbm, pebuf, pesem).wait()

    for i in range(nchunks):
        slot = i % NBUF
        pltpu.make_async_copy(
            x_hbm.at[pl.ds(i * CHUNK, CHUNK), :], xbuf.at[slot], lsem.at[slot]
        ).wait()
        if i >= NBUF:
            j = i - NBUF
            pltpu.make_async_copy(
                obuf.at[slot], o_hbm.at[pl.ds(j * CHUNK, CHUNK), :], ssem.at[slot]
            ).wait()
        pe0 = (i * CHUNK) % s
        obuf[slot] = xbuf[slot] + pebuf[pl.ds(pe0, CHUNK), :]
        pltpu.make_async_copy(
            obuf.at[slot], o_hbm.at[pl.ds(i * CHUNK, CHUNK), :], ssem.at[slot]
        ).start()
        nxt = i + NBUF
        if nxt < nchunks:
            pltpu.make_async_copy(
                x_hbm.at[pl.ds(nxt * CHUNK, CHUNK), :], xbuf.at[slot], lsem.at[slot]
            ).start()

    for i in range(max(0, nchunks - NBUF), nchunks):
        slot = i % NBUF
        pltpu.make_async_copy(
            obuf.at[slot], o_hbm.at[pl.ds(i * CHUNK, CHUNK), :], ssem.at[slot]
        ).wait()


def kernel(x, pe):
    B, S, D = x.shape
    xf = x.reshape(B * S, D)
    out = pl.pallas_call(
        _pe_add_manual,
        in_specs=[
            pl.BlockSpec(memory_space=pl.ANY),
            pl.BlockSpec(memory_space=pl.ANY),
        ],
        out_specs=pl.BlockSpec(memory_space=pl.ANY),
        out_shape=jax.ShapeDtypeStruct((B * S, D), x.dtype),
        scratch_shapes=[
            pltpu.VMEM((NBUF, CHUNK, D), jnp.float32),
            pltpu.VMEM((NBUF, CHUNK, D), jnp.float32),
            pltpu.VMEM((S, D), jnp.float32),
            pltpu.SemaphoreType.DMA((NBUF,)),
            pltpu.SemaphoreType.DMA((NBUF,)),
            pltpu.SemaphoreType.DMA,
        ],
    )(xf, pe[:S])
    return out.reshape(B, S, D)
